# baseline (speedup 1.0000x reference)
import functools

import jax
import jax.numpy as jnp
from jax import lax
from jax.experimental import pallas as pl
from jax.experimental.pallas import tpu as pltpu
from jax.experimental.pallas import tpu_sc as plsc

VOCAB = 100000
HD = 64
HASH = 4096
B = 4096
L = 50
LP = 56
NC = 2
NS = 16
NW = NC * NS
RPW = B // NW
MULT = VOCAB % HASH
MASK = HASH - 1

_mesh = plsc.VectorSubcoreMesh(core_axis_name="c", subcore_axis_name="s")


@functools.partial(
    pl.kernel,
    out_type=jax.ShapeDtypeStruct((B, L, 2 * HD), jnp.float32),
    mesh=_mesh,
    compiler_params=pltpu.CompilerParams(needs_layout_passes=False),
    scratch_types=[
        pltpu.VMEM((RPW, L), jnp.int32),
        pltpu.VMEM((RPW, LP), jnp.int32),
        pltpu.VMEM((RPW, LP), jnp.int32),
        pltpu.VMEM((2, LP, 2 * HD), jnp.float32),
        pltpu.SemaphoreType.DMA((2,)),
        pltpu.SemaphoreType.DMA((2,)),
    ],
)
def _bigram_gather(ids_hbm, u_hbm, b_hbm, out_hbm,
                   ids_v, ui_v, bi_v, comb_v, sem_u, sem_b):
    wid = lax.axis_index("s") * NC + lax.axis_index("c")
    rbase = wid * RPW
    pltpu.sync_copy(ids_hbm.at[pl.ds(rbase, RPW)], ids_v)

    lanes = lax.iota(jnp.int32, 16)
    zeros16 = jnp.zeros((16,), jnp.int32)
    pl1 = jnp.maximum(lanes - 1, 0)

    def fill_body(r, carry):
        ui_v[r, pl.ds(40, 16)] = zeros16
        bi_v[r, pl.ds(40, 16)] = zeros16
        rvec = r + zeros16
        for c, co in ((0, 0), (1, 16), (2, 32), (3, 34)):
            cur = ids_v[r, pl.ds(co, 16)]
            if c == 0:
                prev0 = plsc.load_gather(ids_v, [rvec, pl1])
                pi = jnp.where(lanes == 0, 0, prev0)
            else:
                pi = ids_v[r, pl.ds(co - 1, 16)]
            ui_v[r, pl.ds(co, 16)] = cur
            bi_v[r, pl.ds(co, 16)] = ((pi & MASK) * MULT + cur) & MASK
        return carry

    lax.fori_loop(0, RPW, fill_body, 0)

    def issue(r, p):
        pltpu.async_copy(u_hbm.at[ui_v.at[r]], comb_v.at[p], sem_u.at[p])

    def add_b(r, p):
        pltpu.async_copy(b_hbm.at[bi_v.at[r]], comb_v.at[p], sem_b.at[p],
                         add=True)

    def wait_u(p):
        pltpu.make_async_copy(u_hbm.at[pl.ds(0, LP)], comb_v.at[p],
                              sem_u.at[p]).wait()

    def wait_b(p):
        pltpu.make_async_copy(b_hbm.at[pl.ds(0, LP)], comb_v.at[p],
                              sem_b.at[p]).wait()

    issue(0, 0)

    def g_body(i, carry):
        for (off, p, q) in ((0, 0, 1), (1, 1, 0)):
            r = 2 * i + off

            @pl.when(r + 1 < RPW)
            def _():
                issue(r + 1, q)

            wait_u(p)
            add_b(r, p)
            wait_b(p)
            pltpu.sync_copy(comb_v.at[p, pl.ds(0, L)], out_hbm.at[rbase + r])
        return carry

    lax.fori_loop(0, RPW // 2, g_body, 0)


def kernel(ids, u, b):
    u128 = jnp.pad(u, ((0, 0), (0, HD)))
    b128 = jnp.pad(b, ((0, 0), (HD, 0)))
    return _bigram_gather(ids, u128, b128)
